# baseline (speedup 1.0000x reference)
import functools
import jax
import jax.numpy as jnp
from jax import lax
from jax.experimental import pallas as pl
from jax.experimental.pallas import tpu as pltpu
from jax.experimental.pallas import tpu_sc as plsc

NE = 1024
ED = 256
PTH = 0.8
B, T = 16, 2048
NC, NS, L = 2, 16, 16
NW = NC * NS
HALF = T // 2
CH = 128
NCHUNK = HALF // CH
GP = CH // L
CPR = ED // L
NCOPY = 8


def _sc_body(p_hbm, wp_hbm, out_hbm, part_hbm,
             p_row,
             idx_0, idx_1, pfr_0, pfr_1,
             zp_0, zp_1, ob, accb,
             sg_0, sg_1, so):
  idx = (idx_0, idx_1)
  pfr = (pfr_0, pfr_1)
  zpb = (zp_0, zp_1)

  c = lax.axis_index("c")
  s = lax.axis_index("s")
  wid = s * NC + c
  wid2 = c * NS + s
  b = wid2 // 2
  half = wid2 % 2
  t0 = half * HALF
  row_base = b * T + t0

  pltpu.sync_copy(p_hbm.at[b], p_row)

  iota = lax.iota(jnp.int32, L)
  tbl_off = (wid % NCOPY) * NE

  def _carry_body(i, acc):
    pv = p_row[pl.ds(i * L, L)]
    pos = i * L + iota
    sig = (pv >= PTH) & (pos > 0)
    return acc + jnp.where(sig, 1, 0).astype(jnp.int32)

  carry_vec = lax.fori_loop(0, half * (HALF // L), _carry_body,
                            jnp.zeros((L,), jnp.int32))
  cum = jnp.sum(carry_vec)

  g_cp = [None, None]
  out_cp = [None]

  def build_idx(ch, cum):
    buf = ch % 2
    base = t0 + ch * CH
    for j in range(GP):
      pv = p_row[pl.ds(base + j * L, L)]
      pos = base + j * L + iota
      sig = (pv >= PTH) & (pos > 0)
      sigi = jnp.where(sig, 1, 0).astype(jnp.int32)
      loc = plsc.cumsum(sigi) + cum
      jj = jnp.where(sig, jnp.minimum(loc - 1, NE - 2),
                     jnp.minimum(loc, NE - 1))
      pfs = jnp.where(sig, -pv, pv)
      idx[buf][pl.ds(j * L, L)] = jj + tbl_off
      scat_base = j * (L * L) + iota * L
      for k in range(L):
        plsc.store_scatter(pfr[buf], [scat_base + k], pfs)
      cum = jnp.max(loc)
    return cum

  def blend(ch, acc_s, acc_q):
    buf = ch % 2
    g_cp[buf].wait()
    if out_cp[0] is not None:
      out_cp[0].wait()

    def _blend_body(r, bl_carry):
      a_s, a_q = bl_carry
      pfs = pfr[buf][pl.ds(r * L, L)]
      pf = jnp.abs(pfs)
      sigf = jnp.where(pfs < 0, 1.0, 0.0).astype(jnp.float32)
      for cix in range(CPR):
        zp = plsc.bitcast(zpb[buf][r, pl.ds(cix * L, L)], jnp.bfloat16)
        a, bb = plsc.unpack(zp, format=plsc.PackFormat.INTERLEAVED)
        t = bb - a
        ob[r, pl.ds(cix * L, L)] = a + pf * t
        z1 = a + sigf * t
        a_s = a_s + z1
        a_q = a_q + z1 * z1
      return (a_s, a_q)

    acc_s, acc_q = plsc.parallel_loop(
        0, CH, 1, unroll=1, carry=(acc_s, acc_q))(_blend_body)
    out_cp[0] = pltpu.async_copy(
        ob, out_hbm.at[pl.ds(row_base + ch * CH, CH)], so)
    return acc_s, acc_q

  acc_s = jnp.zeros((L,), jnp.float32)
  acc_q = jnp.zeros((L,), jnp.float32)

  for ch in range(NCHUNK):
    buf = ch % 2
    cum = build_idx(ch, cum)
    g_cp[buf] = pltpu.async_copy(
        wp_hbm.at[idx[buf]], zpb[buf], (sg_0, sg_1)[buf])
    if ch > 0:
      acc_s, acc_q = blend(ch - 1, acc_s, acc_q)
  acc_s, acc_q = blend(NCHUNK - 1, acc_s, acc_q)
  out_cp[0].wait()

  accb[pl.ds(0, L)] = acc_s
  accb[pl.ds(L, L)] = acc_q
  pltpu.sync_copy(accb, part_hbm.at[wid])


_vq3_sc = functools.partial(
    pl.kernel,
    out_type=(jax.ShapeDtypeStruct((B * T, ED), jnp.float32),
              jax.ShapeDtypeStruct((NW, 2 * L), jnp.float32)),
    mesh=plsc.VectorSubcoreMesh(core_axis_name="c", subcore_axis_name="s",
                                num_cores=NC, num_subcores=NS),
    compiler_params=pltpu.CompilerParams(needs_layout_passes=False),
    scratch_types=[
        pltpu.VMEM((T,), jnp.float32),
        pltpu.VMEM((CH,), jnp.int32),
        pltpu.VMEM((CH,), jnp.int32),
        pltpu.VMEM((CH * L,), jnp.float32),
        pltpu.VMEM((CH * L,), jnp.float32),
        pltpu.VMEM((CH, ED), jnp.int32),
        pltpu.VMEM((CH, ED), jnp.int32),
        pltpu.VMEM((CH, ED), jnp.float32),
        pltpu.VMEM((2 * L,), jnp.float32),
        pltpu.SemaphoreType.DMA,
        pltpu.SemaphoreType.DMA,
        pltpu.SemaphoreType.DMA,
    ],
)(_sc_body)


def kernel(p_change, weight):
  wp = jnp.stack([weight[:-1], weight[1:]], axis=-1).astype(jnp.bfloat16)
  wp = lax.bitcast_convert_type(wp, jnp.int32)
  wp = jnp.concatenate([wp] * NCOPY, axis=0)
  z_flat, parts = _vq3_sc(p_change, wp)
  z_out = z_flat.reshape(B, T, ED)
  n = B * T * ED
  ssum = jnp.sum(parts[:, :L])
  qsum = jnp.sum(parts[:, L:])
  v = (qsum - ssum * ssum / n) / (n - 1)
  return (z_out, v)
